# baseline (speedup 1.0000x reference)
import functools

import jax
import jax.numpy as jnp
from jax import lax
from jax.experimental import pallas as pl
from jax.experimental.pallas import tpu as pltpu
from jax.experimental.pallas import tpu_sc as plsc

_NB = 16384
_L = 50
_D = 64
_NC = 2
_NS = 16
_NW = _NC * _NS
_RPW = _NB // _NW
_CR = 16
_N = _RPW // _CR
_NBUF = 2

_mesh = plsc.VectorSubcoreMesh(core_axis_name="c", subcore_axis_name="s")


@functools.partial(
    pl.kernel,
    mesh=_mesh,
    out_type=jax.ShapeDtypeStruct((_NB, _L, _D), jnp.float32),
    scratch_types=(
        [pltpu.VMEM((_CR, _L), jnp.int32) for _ in range(_NBUF)]
        + [pltpu.VMEM((_CR, _L, _D), jnp.float32) for _ in range(_NBUF)]
        + [pltpu.SemaphoreType.DMA for _ in range(2 * _NBUF)]
    ),
    compiler_params=pltpu.CompilerParams(use_tc_tiling_on_sc=False),
)
def _embed_sc(idx_hbm, table_hbm, out_hbm, *scratch):
    idxb = scratch[0:_NBUF]
    rows = scratch[_NBUF:2 * _NBUF]
    gsem = scratch[2 * _NBUF:3 * _NBUF]
    ssem = scratch[3 * _NBUF:4 * _NBUF]

    wid = lax.axis_index("s") * _NC + lax.axis_index("c")
    base = wid * _RPW

    def load_gather(i, b):
        off = base + i * _CR
        pltpu.sync_copy(idx_hbm.at[pl.ds(off, _CR)], idxb[b])
        for j in range(_CR):
            pltpu.async_copy(table_hbm.at[idxb[b].at[j]], rows[b].at[j],
                             gsem[b])

    def wait_gather(b):
        for j in range(_CR):
            pltpu.make_async_copy(table_hbm.at[idxb[b].at[j]], rows[b].at[j],
                                  gsem[b]).wait()

    def start_store(i, b):
        off = base + i * _CR
        pltpu.async_copy(rows[b], out_hbm.at[pl.ds(off, _CR)], ssem[b])

    def wait_store(i, b):
        off = base + i * _CR
        pltpu.make_async_copy(rows[b], out_hbm.at[pl.ds(off, _CR)],
                              ssem[b]).wait()

    for b in range(_NBUF):
        load_gather(b, b)

    def body(g, carry):
        i0 = g * _NBUF
        for b in range(_NBUF):
            wait_gather(b)
            start_store(i0 + b, b)
        for b in range(_NBUF):
            wait_store(i0 + b, b)
            load_gather(i0 + b + _NBUF, b)
        return carry

    lax.fori_loop(0, _N // _NBUF - 1, body, 0)

    i0 = _N - _NBUF
    for b in range(_NBUF):
        wait_gather(b)
        start_store(i0 + b, b)
    for b in range(_NBUF):
        wait_store(i0 + b, b)


def kernel(x, weight):
    if x.dtype != jnp.int32:
        x = x.astype(jnp.int32)
    return _embed_sc(x, weight)
